# baseline (speedup 1.0000x reference)
import functools

import jax
import jax.numpy as jnp
from jax import lax
from jax.experimental import pallas as pl
from jax.experimental.pallas import tpu as pltpu
from jax.experimental.pallas import tpu_sc as plsc

N = 10000
D_IN = 128
D_H = 16

NC = 2
NS = 16
NW = NC * NS
CHUNK = 128
NP = 10112
RPT = NP // NS

_mesh = plsc.VectorSubcoreMesh(core_axis_name="c", subcore_axis_name="s")


def _num_chunks(E):
  ch = -(-E // (NW * CHUNK))
  return -(-ch // 4) * 4


def _make_deg_kernel(CH):
  @functools.partial(
      pl.kernel,
      out_type=jax.ShapeDtypeStruct((NC, NP, D_H), jnp.float32),
      mesh=_mesh,
      compiler_params=pltpu.CompilerParams(use_tc_tiling_on_sc=False),
      scratch_types=[
          pltpu.VMEM((CH, CHUNK), jnp.int32),
          pltpu.VMEM((CH, CHUNK), jnp.int32),
          pltpu.VMEM((CHUNK, D_H), jnp.float32),
          pltpu.VMEM((CHUNK, D_H), jnp.float32),
          pltpu.VMEM_SHARED((NP, D_H), jnp.float32),
          pltpu.SemaphoreType.DMA,
      ],
  )
  def deg_kernel(src_hbm, dst_hbm, esrc_hbm, edst_hbm, zrow_hbm, out_hbm,
                 src_v, dst_v, esrc_v, edst_v, deg_sh, sem):
    c = lax.axis_index("c")
    s = lax.axis_index("s")
    wid = s * NC + c
    srow = s * RPT

    pltpu.sync_copy(src_hbm.at[wid], src_v)
    pltpu.sync_copy(dst_hbm.at[wid], dst_v)
    pltpu.sync_copy(esrc_hbm, esrc_v)
    pltpu.sync_copy(edst_hbm, edst_v)
    pltpu.sync_copy(zrow_hbm.at[pl.ds(srow, RPT)], deg_sh.at[pl.ds(srow, RPT)])
    plsc.subcore_barrier()

    def body(j, carry):
      pltpu.async_copy(esrc_v, deg_sh.at[src_v.at[j]], sem, add=True)
      pltpu.async_copy(edst_v, deg_sh.at[dst_v.at[j]], sem, add=True)

      @pl.when(j >= 4)
      def _():
        pltpu.make_async_copy(esrc_v, deg_sh.at[src_v.at[j]], sem).wait()
        pltpu.make_async_copy(edst_v, deg_sh.at[dst_v.at[j]], sem).wait()

      return carry

    lax.fori_loop(0, CH, body, 0, unroll=False)

    def drain(j, carry):
      pltpu.make_async_copy(esrc_v, deg_sh.at[src_v.at[j]], sem).wait()
      pltpu.make_async_copy(edst_v, deg_sh.at[dst_v.at[j]], sem).wait()
      return carry

    lax.fori_loop(0, 4, drain, 0, unroll=False)
    plsc.subcore_barrier()

    pltpu.sync_copy(deg_sh.at[pl.ds(srow, RPT)], out_hbm.at[c, pl.ds(srow, RPT)])

  return deg_kernel


def _make_agg_kernel(CH):
  @functools.partial(
      pl.kernel,
      out_type=jax.ShapeDtypeStruct((NC, NP, D_H), jnp.float32),
      mesh=_mesh,
      compiler_params=pltpu.CompilerParams(use_tc_tiling_on_sc=False),
      scratch_types=[
          pltpu.VMEM((CH, CHUNK), jnp.int32),
          pltpu.VMEM((CH, CHUNK), jnp.int32),
          [pltpu.VMEM((CHUNK, D_H), jnp.float32)] * 4,
          [pltpu.SemaphoreType.DMA] * 4,
          [pltpu.SemaphoreType.DMA] * 4,
          pltpu.VMEM_SHARED((NP, D_H), jnp.float32),
      ],
  )
  def agg_kernel(src_hbm, dst_hbm, table_hbm, zrows_hbm, out_hbm,
                 src_v, dst_v, bufs, gsems, ssems, agg_sh):
    c = lax.axis_index("c")
    s = lax.axis_index("s")
    wid = s * NC + c
    srow = s * RPT

    pltpu.sync_copy(src_hbm.at[wid], src_v)
    pltpu.sync_copy(dst_hbm.at[wid], dst_v)
    pltpu.sync_copy(zrows_hbm.at[pl.ds(srow, RPT)], agg_sh.at[pl.ds(srow, RPT)])
    plsc.subcore_barrier()

    def gath(j, b):
      pltpu.async_copy(table_hbm.at[src_v.at[j]], bufs[b], gsems[b])

    def gath_wait(j, b):
      pltpu.make_async_copy(table_hbm.at[src_v.at[j]], bufs[b], gsems[b]).wait()

    def scat(j, b):
      pltpu.async_copy(bufs[b], agg_sh.at[dst_v.at[j]], ssems[b], add=True)

    def scat_wait(j, b):
      pltpu.make_async_copy(bufs[b], agg_sh.at[dst_v.at[j]], ssems[b]).wait()

    gath(0, 0)
    gath(1, 1)

    def body(g, carry):
      for u in range(4):
        j = 4 * g + u
        b = u
        b2 = (u + 2) % 4
        gath_wait(j, b)
        scat(j, b)
        if u < 2:
          @pl.when(g > 0)
          def _():
            scat_wait(j - 2, b2)

          gath(j + 2, b2)
        else:
          scat_wait(j - 2, b2)

          @pl.when(g < CH // 4 - 1)
          def _():
            gath(j + 2, b2)
      return carry

    lax.fori_loop(0, CH // 4, body, 0, unroll=False)
    scat_wait(CH - 2, 2)
    scat_wait(CH - 1, 3)
    plsc.subcore_barrier()

    pltpu.sync_copy(agg_sh.at[pl.ds(srow, RPT)], out_hbm.at[c, pl.ds(srow, RPT)])

  return agg_kernel


def _norms(deg_ref):
  dout = deg_ref[0, :, 0:1] + deg_ref[1, :, 0:1]
  din = deg_ref[0, :, 1:2] + deg_ref[1, :, 1:2]
  ns = lax.rsqrt(jnp.maximum(dout, 1.0))
  nd = lax.rsqrt(jnp.maximum(din, 1.0))
  return ns, nd


def _mm1_body(x_ref, w1_ref, deg_ref, o_ref):
  ns, _ = _norms(deg_ref)
  h = jnp.dot(x_ref[...], w1_ref[...], preferred_element_type=jnp.float32)
  o_ref[...] = h * ns


def _mid_body(agg_ref, deg_ref, b1_ref, w2p_ref, o_ref):
  a = agg_ref[0] + agg_ref[1]
  ns, nd = _norms(deg_ref)
  h = jnp.maximum(a * nd + b1_ref[...][None, :], 0.0)
  h2 = jnp.dot(h, w2p_ref[...], preferred_element_type=jnp.float32) * ns
  rows = lax.broadcasted_iota(jnp.int32, (NP, D_H), 0)
  o_ref[...] = jnp.where(rows < N, h2, 0.0)


def _fin_body(agg_ref, deg_ref, b2p_ref, o_ref):
  a = agg_ref[0] + agg_ref[1]
  _, nd = _norms(deg_ref)
  z = a * nd + b2p_ref[...][None, :]
  l0 = z[:, 0:1]
  l1 = z[:, 1:2]
  m = jnp.maximum(l0, l1)
  lse = m + jnp.log(jnp.exp(l0 - m) + jnp.exp(l1 - m))
  o_ref[...] = z - lse


def _tc_call(body, out_shape, *args):
  return pl.pallas_call(
      body, out_shape=jax.ShapeDtypeStruct(out_shape, jnp.float32))(*args)


@jax.jit
def kernel(inputs, edge_index, W1, b1, W2, b2):
  E = edge_index.shape[1]
  CH = _num_chunks(E)
  epad = NW * CH * CHUNK - E

  src = edge_index[0].astype(jnp.int32)
  dst = edge_index[1].astype(jnp.int32)
  fill = jnp.full((epad,), N, dtype=jnp.int32)
  src_slab = jnp.concatenate([src, fill]).reshape(NW, CH, CHUNK)
  dst_slab = jnp.concatenate([dst, fill]).reshape(NW, CH, CHUNK)

  x_pad = jnp.pad(inputs, ((0, NP - N), (0, 0)))
  w2p = jnp.pad(W2, ((0, 0), (0, D_H - W2.shape[1])))
  b2p = jnp.pad(b2, (0, D_H - b2.shape[0]))
  col = jnp.arange(D_H)[None, :]
  e_src = jnp.where(col == 0, 1.0, 0.0).astype(jnp.float32) * jnp.ones((CHUNK, 1), jnp.float32)
  e_dst = jnp.where(col == 1, 1.0, 0.0).astype(jnp.float32) * jnp.ones((CHUNK, 1), jnp.float32)
  zrows = jnp.zeros((NP, D_H), jnp.float32)

  deg = _make_deg_kernel(CH)(src_slab, dst_slab, e_src, e_dst, zrows)

  agg_fn = _make_agg_kernel(CH)

  h1s = _tc_call(_mm1_body, (NP, D_H), x_pad, W1, deg)
  agg1 = agg_fn(src_slab, dst_slab, h1s, zrows)
  h2s = _tc_call(_mid_body, (NP, D_H), agg1, deg, b1, w2p)
  agg2 = agg_fn(src_slab, dst_slab, h2s, zrows)
  out16 = _tc_call(_fin_body, (NP, D_H), agg2, deg, b2p)
  return out16[:N, : W2.shape[1]]


# speedup vs baseline: 1.0931x; 1.0931x over previous
import functools

import jax
import jax.numpy as jnp
from jax import lax
from jax.experimental import pallas as pl
from jax.experimental.pallas import tpu as pltpu
from jax.experimental.pallas import tpu_sc as plsc

N = 10000
D_IN = 128
D_H = 16

NC = 2
NS = 16
NW = NC * NS
CHUNK = 512
D_2 = 8
NP = 10112
RPT = NP // NS

_mesh = plsc.VectorSubcoreMesh(core_axis_name="c", subcore_axis_name="s")


def _num_chunks(E):
  return -(-E // (NW * CHUNK))


def _make_deg_kernel(CH):
  @functools.partial(
      pl.kernel,
      out_type=jax.ShapeDtypeStruct((NC, NP, D_2), jnp.float32),
      mesh=_mesh,
      compiler_params=pltpu.CompilerParams(use_tc_tiling_on_sc=False),
      scratch_types=[
          pltpu.VMEM((CH, CHUNK), jnp.int32),
          pltpu.VMEM((CH, CHUNK), jnp.int32),
          pltpu.VMEM((CHUNK, D_2), jnp.float32),
          pltpu.VMEM((CHUNK, D_2), jnp.float32),
          pltpu.VMEM_SHARED((NP, D_2), jnp.float32),
          pltpu.SemaphoreType.DMA,
      ],
  )
  def deg_kernel(src_hbm, dst_hbm, esrc_hbm, edst_hbm, zrow_hbm, out_hbm,
                 src_v, dst_v, esrc_v, edst_v, deg_sh, sem):
    c = lax.axis_index("c")
    s = lax.axis_index("s")
    wid = s * NC + c
    srow = s * RPT

    pltpu.sync_copy(src_hbm.at[wid], src_v)
    pltpu.sync_copy(dst_hbm.at[wid], dst_v)
    pltpu.sync_copy(esrc_hbm, esrc_v)
    pltpu.sync_copy(edst_hbm, edst_v)
    pltpu.sync_copy(zrow_hbm.at[pl.ds(srow, RPT)], deg_sh.at[pl.ds(srow, RPT)])
    plsc.subcore_barrier()

    pltpu.async_copy(esrc_v, deg_sh.at[src_v.at[0]], sem, add=True)
    pltpu.async_copy(edst_v, deg_sh.at[dst_v.at[0]], sem, add=True)

    def body(j, carry):
      pltpu.async_copy(esrc_v, deg_sh.at[src_v.at[j]], sem, add=True)
      pltpu.make_async_copy(esrc_v, deg_sh.at[src_v.at[j]], sem).wait()
      pltpu.async_copy(edst_v, deg_sh.at[dst_v.at[j]], sem, add=True)
      pltpu.make_async_copy(edst_v, deg_sh.at[dst_v.at[j]], sem).wait()
      return carry

    lax.fori_loop(1, CH, body, 0, unroll=False)
    pltpu.make_async_copy(esrc_v, deg_sh.at[src_v.at[0]], sem).wait()
    pltpu.make_async_copy(edst_v, deg_sh.at[dst_v.at[0]], sem).wait()
    plsc.subcore_barrier()

    pltpu.sync_copy(deg_sh.at[pl.ds(srow, RPT)], out_hbm.at[c, pl.ds(srow, RPT)])

  return deg_kernel


def _make_agg_kernel(CH, D):
  @functools.partial(
      pl.kernel,
      out_type=jax.ShapeDtypeStruct((NC, NP, D), jnp.float32),
      mesh=_mesh,
      compiler_params=pltpu.CompilerParams(use_tc_tiling_on_sc=False),
      scratch_types=[
          pltpu.VMEM((CH, CHUNK), jnp.int32),
          pltpu.VMEM((CH, CHUNK), jnp.int32),
          pltpu.VMEM((CHUNK, D), jnp.float32),
          pltpu.VMEM((CHUNK, D), jnp.float32),
          pltpu.VMEM_SHARED((NP, D), jnp.float32),
          pltpu.SemaphoreType.DMA,
          pltpu.SemaphoreType.DMA,
      ],
  )
  def agg_kernel(src_hbm, dst_hbm, table_hbm, zrows_hbm, out_hbm,
                 src_v, dst_v, buf0, buf1, agg_sh, sem0, sem1):
    c = lax.axis_index("c")
    s = lax.axis_index("s")
    wid = s * NC + c
    srow = s * RPT

    pltpu.sync_copy(src_hbm.at[wid], src_v)
    pltpu.sync_copy(dst_hbm.at[wid], dst_v)
    pltpu.sync_copy(zrows_hbm.at[pl.ds(srow, RPT)], agg_sh.at[pl.ds(srow, RPT)])
    plsc.subcore_barrier()

    pltpu.async_copy(table_hbm.at[src_v.at[0]], buf0, sem0)

    def body(g, carry):
      j = 2 * g

      @pl.when(j + 1 < CH)
      def _():
        pltpu.async_copy(table_hbm.at[src_v.at[j + 1]], buf1, sem1)

      pltpu.make_async_copy(table_hbm.at[src_v.at[j]], buf0, sem0).wait()
      pltpu.sync_copy(buf0, agg_sh.at[dst_v.at[j]], add=True)

      @pl.when(j + 2 < CH)
      def _():
        pltpu.async_copy(table_hbm.at[src_v.at[j + 2]], buf0, sem0)

      @pl.when(j + 1 < CH)
      def _():
        pltpu.make_async_copy(table_hbm.at[src_v.at[j + 1]], buf1, sem1).wait()
        pltpu.sync_copy(buf1, agg_sh.at[dst_v.at[j + 1]], add=True)

      return carry

    lax.fori_loop(0, (CH + 1) // 2, body, 0, unroll=False)
    plsc.subcore_barrier()

    pltpu.sync_copy(agg_sh.at[pl.ds(srow, RPT)], out_hbm.at[c, pl.ds(srow, RPT)])

  return agg_kernel


def _norms(deg_ref):
  dout = deg_ref[0, :, 0:1] + deg_ref[1, :, 0:1]
  din = deg_ref[0, :, 1:2] + deg_ref[1, :, 1:2]
  ns = lax.rsqrt(jnp.maximum(dout, 1.0))
  nd = lax.rsqrt(jnp.maximum(din, 1.0))
  return ns, nd


def _mm1_body(x_ref, w1_ref, deg_ref, o_ref):
  ns, _ = _norms(deg_ref)
  h = jnp.dot(x_ref[...], w1_ref[...], preferred_element_type=jnp.float32)
  o_ref[...] = h * ns


def _mid_body(agg_ref, deg_ref, b1_ref, w2p_ref, o_ref):
  a = agg_ref[0] + agg_ref[1]
  ns, nd = _norms(deg_ref)
  h = jnp.maximum(a * nd + b1_ref[...][None, :], 0.0)
  h2 = jnp.dot(h, w2p_ref[...], preferred_element_type=jnp.float32) * ns
  rows = lax.broadcasted_iota(jnp.int32, (NP, D_2), 0)
  o_ref[...] = jnp.where(rows < N, h2, 0.0)


def _fin_body(agg_ref, deg_ref, b2p_ref, o_ref):
  a = agg_ref[0] + agg_ref[1]
  _, nd = _norms(deg_ref)
  z = a * nd + b2p_ref[...][None, :]
  l0 = z[:, 0:1]
  l1 = z[:, 1:2]
  m = jnp.maximum(l0, l1)
  lse = m + jnp.log(jnp.exp(l0 - m) + jnp.exp(l1 - m))
  o_ref[...] = z - lse


def _tc_call(body, out_shape, *args):
  return pl.pallas_call(
      body, out_shape=jax.ShapeDtypeStruct(out_shape, jnp.float32))(*args)


@jax.jit
def kernel(inputs, edge_index, W1, b1, W2, b2):
  E = edge_index.shape[1]
  CH = _num_chunks(E)
  epad = NW * CH * CHUNK - E

  src = edge_index[0].astype(jnp.int32)
  dst = edge_index[1].astype(jnp.int32)
  fill = jnp.full((epad,), N, dtype=jnp.int32)
  src_slab = jnp.concatenate([src, fill]).reshape(NW, CH, CHUNK)
  dst_slab = jnp.concatenate([dst, fill]).reshape(NW, CH, CHUNK)

  x_pad = jnp.pad(inputs, ((0, NP - N), (0, 0)))
  w2p = jnp.pad(W2, ((0, 0), (0, D_2 - W2.shape[1])))
  b2p = jnp.pad(b2, (0, D_2 - b2.shape[0]))
  col = jnp.arange(D_2)[None, :]
  e_src = jnp.where(col == 0, 1.0, 0.0).astype(jnp.float32) * jnp.ones((CHUNK, 1), jnp.float32)
  e_dst = jnp.where(col == 1, 1.0, 0.0).astype(jnp.float32) * jnp.ones((CHUNK, 1), jnp.float32)
  zrows16 = jnp.zeros((NP, D_H), jnp.float32)
  zrows8 = jnp.zeros((NP, D_2), jnp.float32)

  deg = _make_deg_kernel(CH)(src_slab, dst_slab, e_src, e_dst, zrows8)

  h1s = _tc_call(_mm1_body, (NP, D_H), x_pad, W1, deg)
  agg1 = _make_agg_kernel(CH, D_H)(src_slab, dst_slab, h1s, zrows16)
  h2s = _tc_call(_mid_body, (NP, D_2), agg1, deg, b1, w2p)
  agg2 = _make_agg_kernel(CH, D_2)(src_slab, dst_slab, h2s, zrows8)
  out8 = _tc_call(_fin_body, (NP, D_2), agg2, deg, b2p)
  return out8[:N, : W2.shape[1]]


# speedup vs baseline: 1.5781x; 1.4437x over previous
import functools

import jax
import jax.numpy as jnp
from jax import lax
from jax.experimental import pallas as pl
from jax.experimental.pallas import tpu as pltpu
from jax.experimental.pallas import tpu_sc as plsc

N = 10000
D_IN = 128
D_H = 16

NC = 2
NS = 16
NW = NC * NS
CHUNK = 512
D_2 = 8
NP = 10112
RPT = NP // NS

_mesh = plsc.VectorSubcoreMesh(core_axis_name="c", subcore_axis_name="s")


def _num_chunks(E):
  return -(-E // (NW * CHUNK))


def _make_deg_kernel(CH):
  @functools.partial(
      pl.kernel,
      out_type=jax.ShapeDtypeStruct((NC, NP, D_2), jnp.float32),
      mesh=_mesh,
      compiler_params=pltpu.CompilerParams(use_tc_tiling_on_sc=False),
      scratch_types=[
          pltpu.VMEM((CH, CHUNK), jnp.int32),
          pltpu.VMEM((CH, CHUNK), jnp.int32),
          pltpu.VMEM((CHUNK, D_2), jnp.float32),
          pltpu.VMEM((CHUNK, D_2), jnp.float32),
          pltpu.VMEM_SHARED((NP, D_2), jnp.float32),
          pltpu.SemaphoreType.DMA,
      ],
  )
  def deg_kernel(src_hbm, dst_hbm, esrc_hbm, edst_hbm, zrow_hbm, out_hbm,
                 src_v, dst_v, esrc_v, edst_v, deg_sh, sem):
    c = lax.axis_index("c")
    s = lax.axis_index("s")
    wid = s * NC + c
    srow = s * RPT

    pltpu.sync_copy(src_hbm.at[wid], src_v)
    pltpu.sync_copy(dst_hbm.at[wid], dst_v)
    pltpu.sync_copy(esrc_hbm, esrc_v)
    pltpu.sync_copy(edst_hbm, edst_v)
    pltpu.sync_copy(zrow_hbm.at[pl.ds(srow, RPT)], deg_sh.at[pl.ds(srow, RPT)])
    plsc.subcore_barrier()

    pltpu.async_copy(esrc_v, deg_sh.at[src_v.at[0]], sem, add=True)
    pltpu.async_copy(edst_v, deg_sh.at[dst_v.at[0]], sem, add=True)

    def body(j, carry):
      pltpu.async_copy(esrc_v, deg_sh.at[src_v.at[j]], sem, add=True)
      pltpu.make_async_copy(esrc_v, deg_sh.at[src_v.at[j]], sem).wait()
      pltpu.async_copy(edst_v, deg_sh.at[dst_v.at[j]], sem, add=True)
      pltpu.make_async_copy(edst_v, deg_sh.at[dst_v.at[j]], sem).wait()
      return carry

    lax.fori_loop(1, CH, body, 0, unroll=False)
    pltpu.make_async_copy(esrc_v, deg_sh.at[src_v.at[0]], sem).wait()
    pltpu.make_async_copy(edst_v, deg_sh.at[dst_v.at[0]], sem).wait()
    plsc.subcore_barrier()

    pltpu.sync_copy(deg_sh.at[pl.ds(srow, RPT)], out_hbm.at[c, pl.ds(srow, RPT)])

  return deg_kernel


def _make_agg_kernel(CH, D):
  @functools.partial(
      pl.kernel,
      out_type=jax.ShapeDtypeStruct((NC, NP, D), jnp.float32),
      mesh=_mesh,
      compiler_params=pltpu.CompilerParams(use_tc_tiling_on_sc=False),
      scratch_types=[
          pltpu.VMEM((CH, CHUNK), jnp.int32),
          pltpu.VMEM((CH, CHUNK), jnp.int32),
          pltpu.VMEM((CHUNK, D), jnp.float32),
          pltpu.VMEM((CHUNK, D), jnp.float32),
          pltpu.VMEM_SHARED((NP, D), jnp.float32),
          pltpu.SemaphoreType.DMA,
          pltpu.SemaphoreType.DMA,
      ],
  )
  def agg_kernel(src_hbm, dst_hbm, table_hbm, zrows_hbm, out_hbm,
                 src_v, dst_v, buf0, buf1, agg_sh, sem0, sem1):
    c = lax.axis_index("c")
    s = lax.axis_index("s")
    wid = s * NC + c
    srow = s * RPT

    pltpu.sync_copy(src_hbm.at[wid], src_v)
    pltpu.sync_copy(dst_hbm.at[wid], dst_v)
    pltpu.sync_copy(zrows_hbm.at[pl.ds(srow, RPT)], agg_sh.at[pl.ds(srow, RPT)])
    plsc.subcore_barrier()

    pltpu.async_copy(table_hbm.at[src_v.at[0]], buf0, sem0)

    def body(g, carry):
      j = 2 * g

      @pl.when(j + 1 < CH)
      def _():
        pltpu.async_copy(table_hbm.at[src_v.at[j + 1]], buf1, sem1)

      pltpu.make_async_copy(table_hbm.at[src_v.at[j]], buf0, sem0).wait()
      pltpu.sync_copy(buf0, agg_sh.at[dst_v.at[j]], add=True)

      @pl.when(j + 2 < CH)
      def _():
        pltpu.async_copy(table_hbm.at[src_v.at[j + 2]], buf0, sem0)

      @pl.when(j + 1 < CH)
      def _():
        pltpu.make_async_copy(table_hbm.at[src_v.at[j + 1]], buf1, sem1).wait()
        pltpu.sync_copy(buf1, agg_sh.at[dst_v.at[j + 1]], add=True)

      return carry

    lax.fori_loop(0, (CH + 1) // 2, body, 0, unroll=False)
    plsc.subcore_barrier()

    pltpu.sync_copy(agg_sh.at[pl.ds(srow, RPT)], out_hbm.at[c, pl.ds(srow, RPT)])

  return agg_kernel


def _norms(deg_ref):
  dout = deg_ref[0, :, 0:1] + deg_ref[1, :, 0:1]
  din = deg_ref[0, :, 1:2] + deg_ref[1, :, 1:2]
  ns = lax.rsqrt(jnp.maximum(dout, 1.0))
  nd = lax.rsqrt(jnp.maximum(din, 1.0))
  return ns, nd


def _mm1_body(x_ref, w1_ref, deg_ref, o_ref):
  ns, _ = _norms(deg_ref)
  h = jnp.dot(x_ref[...], w1_ref[...], preferred_element_type=jnp.float32)
  o_ref[...] = h * ns


def _mid_body(agg_ref, deg_ref, b1_ref, w2p_ref, o_ref):
  a = agg_ref[0] + agg_ref[1]
  ns, nd = _norms(deg_ref)
  h = jnp.maximum(a * nd + b1_ref[...][None, :], 0.0)
  h2 = jnp.dot(h, w2p_ref[...], preferred_element_type=jnp.float32) * ns
  rows = lax.broadcasted_iota(jnp.int32, (NP, D_2), 0)
  o_ref[...] = jnp.where(rows < N, h2, 0.0)


def _fin_body(agg_ref, deg_ref, b2p_ref, o_ref):
  a = agg_ref[0] + agg_ref[1]
  _, nd = _norms(deg_ref)
  z = a * nd + b2p_ref[...][None, :]
  l0 = z[:, 0:1]
  l1 = z[:, 1:2]
  m = jnp.maximum(l0, l1)
  lse = m + jnp.log(jnp.exp(l0 - m) + jnp.exp(l1 - m))
  o_ref[...] = z - lse


def _tc_call(body, out_shape, *args):
  return pl.pallas_call(
      body, out_shape=jax.ShapeDtypeStruct(out_shape, jnp.float32))(*args)


@jax.jit
def kernel(inputs, edge_index, W1, b1, W2, b2):
  E = edge_index.shape[1]
  CH = _num_chunks(E)
  epad = NW * CH * CHUNK - E

  src = edge_index[0].astype(jnp.int32)
  dst = edge_index[1].astype(jnp.int32)
  fill = (N + jnp.arange(epad, dtype=jnp.int32) % (NP - N)).astype(jnp.int32)
  src_slab = jnp.concatenate([src, fill]).reshape(NW, CH, CHUNK)
  dst_slab = jnp.concatenate([dst, fill]).reshape(NW, CH, CHUNK)

  x_pad = jnp.pad(inputs, ((0, NP - N), (0, 0)))
  w2p = jnp.pad(W2, ((0, 0), (0, D_2 - W2.shape[1])))
  b2p = jnp.pad(b2, (0, D_2 - b2.shape[0]))
  col = jnp.arange(D_2)[None, :]
  e_src = jnp.where(col == 0, 1.0, 0.0).astype(jnp.float32) * jnp.ones((CHUNK, 1), jnp.float32)
  e_dst = jnp.where(col == 1, 1.0, 0.0).astype(jnp.float32) * jnp.ones((CHUNK, 1), jnp.float32)
  zrows16 = jnp.zeros((NP, D_H), jnp.float32)
  zrows8 = jnp.zeros((NP, D_2), jnp.float32)

  deg = _make_deg_kernel(CH)(src_slab, dst_slab, e_src, e_dst, zrows8)

  h1s = _tc_call(_mm1_body, (NP, D_H), x_pad, W1, deg)
  agg1 = _make_agg_kernel(CH, D_H)(src_slab, dst_slab, h1s, zrows16)
  h2s = _tc_call(_mid_body, (NP, D_2), agg1, deg, b1, w2p)
  agg2 = _make_agg_kernel(CH, D_2)(src_slab, dst_slab, h2s, zrows8)
  out8 = _tc_call(_fin_body, (NP, D_2), agg2, deg, b2p)
  return out8[:N, : W2.shape[1]]


# speedup vs baseline: 1.6016x; 1.0149x over previous
import functools

import jax
import jax.numpy as jnp
from jax import lax
from jax.experimental import pallas as pl
from jax.experimental.pallas import tpu as pltpu
from jax.experimental.pallas import tpu_sc as plsc

N = 10000
D_IN = 128
D_H = 16

NC = 2
NS = 16
NW = NC * NS
CHUNK = 512
D_2 = 8
NP = 10112
RPT = NP // NS

_mesh = plsc.VectorSubcoreMesh(core_axis_name="c", subcore_axis_name="s")


def _tile_chunks(wid, nch):
  base = nch // NW
  rem = nch - base * NW
  extra = jnp.maximum(wid - (NW - rem), 0)
  start = base * wid + extra
  num = base + jnp.where(wid >= NW - rem, 1, 0)
  return start, num


def _make_deg_kernel(nch, chmax):
  @functools.partial(
      pl.kernel,
      out_type=jax.ShapeDtypeStruct((NC, NP, D_2), jnp.float32),
      mesh=_mesh,
      compiler_params=pltpu.CompilerParams(use_tc_tiling_on_sc=False),
      scratch_types=[
          pltpu.VMEM((chmax, CHUNK), jnp.int32),
          pltpu.VMEM((chmax, CHUNK), jnp.int32),
          pltpu.VMEM((CHUNK, D_2), jnp.float32),
          pltpu.VMEM((CHUNK, D_2), jnp.float32),
          pltpu.VMEM_SHARED((NP, D_2), jnp.float32),
          pltpu.SemaphoreType.DMA,
      ],
  )
  def deg_kernel(src_hbm, dst_hbm, esrc_hbm, edst_hbm, zrow_hbm, out_hbm,
                 src_v, dst_v, esrc_v, edst_v, deg_sh, sem):
    c = lax.axis_index("c")
    s = lax.axis_index("s")
    wid = s * NC + c
    srow = s * RPT
    start, num = _tile_chunks(wid, nch)

    pltpu.sync_copy(src_hbm.at[pl.ds(start, chmax)], src_v)
    pltpu.sync_copy(dst_hbm.at[pl.ds(start, chmax)], dst_v)
    pltpu.sync_copy(esrc_hbm, esrc_v)
    pltpu.sync_copy(edst_hbm, edst_v)
    pltpu.sync_copy(zrow_hbm.at[pl.ds(srow, RPT)], deg_sh.at[pl.ds(srow, RPT)])
    plsc.subcore_barrier()

    pltpu.async_copy(esrc_v, deg_sh.at[src_v.at[0]], sem, add=True)
    pltpu.async_copy(edst_v, deg_sh.at[dst_v.at[0]], sem, add=True)

    def body(j, carry):
      @pl.when(j < num)
      def _():
        pltpu.async_copy(esrc_v, deg_sh.at[src_v.at[j]], sem, add=True)
        pltpu.make_async_copy(esrc_v, deg_sh.at[src_v.at[j]], sem).wait()
        pltpu.async_copy(edst_v, deg_sh.at[dst_v.at[j]], sem, add=True)
        pltpu.make_async_copy(edst_v, deg_sh.at[dst_v.at[j]], sem).wait()

      return carry

    lax.fori_loop(1, chmax, body, 0, unroll=False)
    pltpu.make_async_copy(esrc_v, deg_sh.at[src_v.at[0]], sem).wait()
    pltpu.make_async_copy(edst_v, deg_sh.at[dst_v.at[0]], sem).wait()
    plsc.subcore_barrier()

    pltpu.sync_copy(deg_sh.at[pl.ds(srow, RPT)], out_hbm.at[c, pl.ds(srow, RPT)])

  return deg_kernel


def _make_agg_kernel(nch, chmax, D):
  @functools.partial(
      pl.kernel,
      out_type=jax.ShapeDtypeStruct((NC, NP, D), jnp.float32),
      mesh=_mesh,
      compiler_params=pltpu.CompilerParams(use_tc_tiling_on_sc=False),
      scratch_types=[
          pltpu.VMEM((chmax, CHUNK), jnp.int32),
          pltpu.VMEM((chmax, CHUNK), jnp.int32),
          pltpu.VMEM((CHUNK, D), jnp.float32),
          pltpu.VMEM((CHUNK, D), jnp.float32),
          pltpu.VMEM_SHARED((NP, D), jnp.float32),
          pltpu.SemaphoreType.DMA,
          pltpu.SemaphoreType.DMA,
      ],
  )
  def agg_kernel(src_hbm, dst_hbm, table_hbm, zrows_hbm, out_hbm,
                 src_v, dst_v, buf0, buf1, agg_sh, sem0, sem1):
    c = lax.axis_index("c")
    s = lax.axis_index("s")
    wid = s * NC + c
    srow = s * RPT
    start, num = _tile_chunks(wid, nch)

    pltpu.sync_copy(src_hbm.at[pl.ds(start, chmax)], src_v)
    pltpu.sync_copy(dst_hbm.at[pl.ds(start, chmax)], dst_v)
    pltpu.sync_copy(zrows_hbm.at[pl.ds(srow, RPT)], agg_sh.at[pl.ds(srow, RPT)])
    plsc.subcore_barrier()

    pltpu.async_copy(table_hbm.at[src_v.at[0]], buf0, sem0)

    def body(g, carry):
      j = 2 * g

      @pl.when(j + 1 < num)
      def _():
        pltpu.async_copy(table_hbm.at[src_v.at[j + 1]], buf1, sem1)

      pltpu.make_async_copy(table_hbm.at[src_v.at[j]], buf0, sem0).wait()
      pltpu.sync_copy(buf0, agg_sh.at[dst_v.at[j]], add=True)

      @pl.when(j + 2 < num)
      def _():
        pltpu.async_copy(table_hbm.at[src_v.at[j + 2]], buf0, sem0)

      @pl.when(j + 1 < num)
      def _():
        pltpu.make_async_copy(table_hbm.at[src_v.at[j + 1]], buf1, sem1).wait()
        pltpu.sync_copy(buf1, agg_sh.at[dst_v.at[j + 1]], add=True)

      return carry

    lax.fori_loop(0, (chmax + 1) // 2, body, 0, unroll=False)
    plsc.subcore_barrier()

    pltpu.sync_copy(agg_sh.at[pl.ds(srow, RPT)], out_hbm.at[c, pl.ds(srow, RPT)])

  return agg_kernel


def _norms(deg_ref):
  dout = deg_ref[0, 0:N, 0:1] + deg_ref[1, 0:N, 0:1]
  din = deg_ref[0, 0:N, 1:2] + deg_ref[1, 0:N, 1:2]
  ns = lax.rsqrt(jnp.maximum(dout, 1.0))
  nd = lax.rsqrt(jnp.maximum(din, 1.0))
  return ns, nd


def _mm1_body(x_ref, w1_ref, deg_ref, o_ref):
  ns, _ = _norms(deg_ref)
  h = jnp.dot(x_ref[...], w1_ref[...], preferred_element_type=jnp.float32)
  o_ref[...] = h * ns


def _mid_body(agg_ref, deg_ref, b1_ref, w2p_ref, o_ref):
  a = agg_ref[0, 0:N] + agg_ref[1, 0:N]
  ns, nd = _norms(deg_ref)
  h = jnp.maximum(a * nd + b1_ref[...][None, :], 0.0)
  o_ref[...] = jnp.dot(h, w2p_ref[...], preferred_element_type=jnp.float32) * ns


def _fin_body(agg_ref, deg_ref, b2_ref, o_ref):
  a = agg_ref[0, 0:N, 0:2] + agg_ref[1, 0:N, 0:2]
  _, nd = _norms(deg_ref)
  z = a * nd + b2_ref[...][None, :]
  l0 = z[:, 0:1]
  l1 = z[:, 1:2]
  m = jnp.maximum(l0, l1)
  lse = m + jnp.log(jnp.exp(l0 - m) + jnp.exp(l1 - m))
  o_ref[...] = z - lse


def _tc_call(body, out_shape, *args):
  return pl.pallas_call(
      body, out_shape=jax.ShapeDtypeStruct(out_shape, jnp.float32))(*args)


@jax.jit
def kernel(inputs, edge_index, W1, b1, W2, b2):
  E = edge_index.shape[1]
  assert E % CHUNK == 0
  nch = E // CHUNK
  chmax = nch // NW + (1 if nch % NW else 0)

  src_chunks = edge_index[0].astype(jnp.int32).reshape(nch, CHUNK)
  dst_chunks = edge_index[1].astype(jnp.int32).reshape(nch, CHUNK)

  w2p = jnp.pad(W2, ((0, 0), (0, D_2 - W2.shape[1])))
  col = jnp.arange(D_2)[None, :]
  e_src = jnp.where(col == 0, 1.0, 0.0).astype(jnp.float32) * jnp.ones((CHUNK, 1), jnp.float32)
  e_dst = jnp.where(col == 1, 1.0, 0.0).astype(jnp.float32) * jnp.ones((CHUNK, 1), jnp.float32)
  zrows16 = jnp.zeros((NP, D_H), jnp.float32)
  zrows8 = jnp.zeros((NP, D_2), jnp.float32)

  deg = _make_deg_kernel(nch, chmax)(src_chunks, dst_chunks, e_src, e_dst, zrows8)

  h1s = _tc_call(_mm1_body, (N, D_H), inputs, W1, deg)
  agg1 = _make_agg_kernel(nch, chmax, D_H)(src_chunks, dst_chunks, h1s, zrows16)
  h2s = _tc_call(_mid_body, (N, D_2), agg1, deg, b1, w2p)
  agg2 = _make_agg_kernel(nch, chmax, D_2)(src_chunks, dst_chunks, h2s, zrows8)
  return _tc_call(_fin_body, (N, 2), agg2, deg, b2)
